# Initial kernel scaffold; baseline (speedup 1.0000x reference)
import jax
import jax.numpy as jnp
from jax.experimental import pallas as pl


def kernel(x, edge_index, edge_attr, batch, W0, b0, We1, be1, We2, be2, bconv, Wih_g, Whh_g, bih_g, bhh_g, Wih_l, Whh_l, bih_l, bhh_l):
    raise NotImplementedError("write your pallas kernel here")


# speedup vs baseline: 3.6653x; 3.6653x over previous
import functools

import jax
import jax.numpy as jnp
from jax import lax
from jax.experimental import pallas as pl
from jax.experimental.pallas import tpu as pltpu
from jax.experimental.pallas import tpu_sc as plsc

N = 10000
E = 160000
F_IN = 128
DIM = 32
EA = 16
B = 64

NC = 2
NS = 16
NW = NC * NS
PER_W = E // NW
CH = 1000
NCH = PER_W // CH


def _gather_body(feat_hbm, src_hbm, xj_hbm, idx_v, rows0, rows1, sem0, sem1):
    wid = lax.axis_index("s") * NC + lax.axis_index("c")
    base0 = wid * PER_W
    pltpu.sync_copy(src_hbm.at[pl.ds(base0, PER_W)], idx_v)
    rows = (rows0, rows1)
    sems = (sem0, sem1)
    pending = {0: pltpu.async_copy(
        feat_hbm.at[idx_v.at[pl.ds(0, CH)]], rows0, sem0)}
    for c in range(NCH):
        cur = c % 2
        if c + 1 < NCH:
            nb = (c + 1) % 2
            pending[c + 1] = pltpu.async_copy(
                feat_hbm.at[idx_v.at[pl.ds((c + 1) * CH, CH)]], rows[nb],
                sems[nb])
        pending.pop(c).wait()
        pltpu.sync_copy(rows[cur], xj_hbm.at[pl.ds(base0 + c * CH, CH)])


def _make_scatter_body(with_deg):
    def body(*refs):
        if with_deg:
            (msg_hbm, dst_hbm, ones_hbm, zero_hbm, zeron_hbm, part_hbm,
             degp_hbm, idx0, idx1, rows0, rows1, ones_v, acc_sh, deg_sh,
             sem0, sem1) = refs
        else:
            (msg_hbm, dst_hbm, zero_hbm, part_hbm,
             idx0, idx1, rows0, rows1, acc_sh, sem0, sem1) = refs
        c = lax.axis_index("c")
        s = lax.axis_index("s")

        @pl.when(s == 0)
        def _():
            pltpu.sync_copy(zero_hbm, acc_sh)

        if with_deg:
            pltpu.sync_copy(ones_hbm, ones_v)

            @pl.when(s == 0)
            def _():
                pltpu.sync_copy(zeron_hbm, deg_sh)

        plsc.subcore_barrier()
        base0 = (s * NC + c) * PER_W
        idx = (idx0, idx1)
        rows = (rows0, rows1)
        sems = (sem0, sem1)
        pending = {0: (
            pltpu.async_copy(dst_hbm.at[pl.ds(base0, CH)], idx0, sem0),
            pltpu.async_copy(msg_hbm.at[pl.ds(base0, CH)], rows0, sem0),
        )}
        for ch in range(NCH):
            cur = ch % 2
            if ch + 1 < NCH:
                nb = (ch + 1) % 2
                off = base0 + (ch + 1) * CH
                pending[ch + 1] = (
                    pltpu.async_copy(dst_hbm.at[pl.ds(off, CH)], idx[nb],
                                     sems[nb]),
                    pltpu.async_copy(msg_hbm.at[pl.ds(off, CH)], rows[nb],
                                     sems[nb]),
                )
            di, dr = pending.pop(ch)
            di.wait()
            dr.wait()
            pltpu.sync_copy(rows[cur], acc_sh.at[idx[cur]], add=True)
            if with_deg:
                pltpu.sync_copy(ones_v, deg_sh.at[idx[cur]], add=True)
        plsc.subcore_barrier()

        @pl.when(s == 0)
        def _():
            pltpu.sync_copy(acc_sh, part_hbm.at[c])

        if with_deg:
            @pl.when(s == 0)
            def _():
                pltpu.sync_copy(deg_sh, degp_hbm.at[c])
    return body


@functools.cache
def _sc_kernels():
    mesh = plsc.VectorSubcoreMesh(core_axis_name="c", subcore_axis_name="s")
    params = pltpu.CompilerParams(use_tc_tiling_on_sc=False)
    gather = pl.kernel(
        _gather_body,
        out_type=jax.ShapeDtypeStruct((E, DIM), jnp.float32),
        mesh=mesh,
        compiler_params=params,
        scratch_types=[
            pltpu.VMEM((PER_W,), jnp.int32),
            pltpu.VMEM((CH, DIM), jnp.float32),
            pltpu.VMEM((CH, DIM), jnp.float32),
            pltpu.SemaphoreType.DMA,
            pltpu.SemaphoreType.DMA,
        ],
    )
    scatter_bufs = [
        pltpu.VMEM((CH,), jnp.int32),
        pltpu.VMEM((CH,), jnp.int32),
        pltpu.VMEM((CH, DIM), jnp.float32),
        pltpu.VMEM((CH, DIM), jnp.float32),
    ]
    scatter = pl.kernel(
        _make_scatter_body(False),
        out_type=jax.ShapeDtypeStruct((NC, N, DIM), jnp.float32),
        mesh=mesh,
        compiler_params=params,
        scratch_types=scatter_bufs + [
            pltpu.VMEM_SHARED((N, DIM), jnp.float32),
            pltpu.SemaphoreType.DMA,
            pltpu.SemaphoreType.DMA,
        ],
    )
    scatter_deg = pl.kernel(
        _make_scatter_body(True),
        out_type=[jax.ShapeDtypeStruct((NC, N, DIM), jnp.float32),
                  jax.ShapeDtypeStruct((NC, N), jnp.float32)],
        mesh=mesh,
        compiler_params=params,
        scratch_types=scatter_bufs + [
            pltpu.VMEM((CH,), jnp.float32),
            pltpu.VMEM_SHARED((N, DIM), jnp.float32),
            pltpu.VMEM_SHARED((N,), jnp.float32),
            pltpu.SemaphoreType.DMA,
            pltpu.SemaphoreType.DMA,
        ],
    )
    return gather, scatter, scatter_deg


def _gather_call(feat, src):
    return _sc_kernels()[0](feat, src)


def _scatter_call(msg, dst, zeros_nd):
    return _sc_kernels()[1](msg, dst, zeros_nd)


def _scatter_deg_call(msg, dst, ones_ch, zeros_nd, zeros_n):
    return _sc_kernels()[2](msg, dst, ones_ch, zeros_nd, zeros_n)


TN_PROJ = 2000


def _proj_body(x_ref, w_ref, b_ref, o_ref):
    o_ref[...] = jax.nn.relu(
        jnp.dot(x_ref[...], w_ref[...], preferred_element_type=jnp.float32)
        + b_ref[...]
    )


def _proj(x, w0t, b0r):
    return pl.pallas_call(
        _proj_body,
        grid=(N // TN_PROJ,),
        in_specs=[
            pl.BlockSpec((TN_PROJ, F_IN), lambda i: (i, 0)),
            pl.BlockSpec((F_IN, DIM), lambda i: (0, 0)),
            pl.BlockSpec((1, DIM), lambda i: (0, 0)),
        ],
        out_specs=pl.BlockSpec((TN_PROJ, DIM), lambda i: (i, 0)),
        out_shape=jax.ShapeDtypeStruct((N, DIM), jnp.float32),
    )(x, w0t, b0r)


TE = 2000


def _msg_body(ea_ref, xj_ref, w1_ref, b1_ref, w2_ref, b2_ref, r_ref, o_ref):
    hid = jax.nn.relu(
        jnp.dot(ea_ref[...], w1_ref[...], preferred_element_type=jnp.float32)
        + b1_ref[...]
    )
    wm = jnp.dot(hid.astype(jnp.bfloat16), w2_ref[...],
                 preferred_element_type=jnp.float32).astype(jnp.bfloat16)
    wm = wm + b2_ref[...]
    xjr = jnp.dot(xj_ref[...].astype(jnp.bfloat16), r_ref[...],
                  preferred_element_type=jnp.float32).astype(jnp.bfloat16)
    p = wm * xjr
    w = DIM * DIM
    while w > DIM:
        w //= 2
        p = p[:, :w] + p[:, w:]
    o_ref[...] = p.astype(jnp.float32)


def _msg(edge_attr, xj, w1t, b1r, w2t, b2r, rmat):
    return pl.pallas_call(
        _msg_body,
        grid=(E // TE,),
        in_specs=[
            pl.BlockSpec((TE, EA), lambda i: (i, 0)),
            pl.BlockSpec((TE, DIM), lambda i: (i, 0)),
            pl.BlockSpec((EA, F_IN), lambda i: (0, 0)),
            pl.BlockSpec((1, F_IN), lambda i: (0, 0)),
            pl.BlockSpec((F_IN, DIM * DIM), lambda i: (0, 0)),
            pl.BlockSpec((1, DIM * DIM), lambda i: (0, 0)),
            pl.BlockSpec((DIM, DIM * DIM), lambda i: (0, 0)),
        ],
        out_specs=pl.BlockSpec((TE, DIM), lambda i: (i, 0)),
        out_shape=jax.ShapeDtypeStruct((E, DIM), jnp.float32),
    )(edge_attr, xj, w1t, b1r, w2t, b2r, rmat)


TN_GRU = 2000


def _gru_body(p0_ref, p1_ref, d0_ref, d1_ref, h_ref, bc_ref, wi_ref, bi_ref,
              wh_ref, bh_ref, o_ref):
    deg = jnp.maximum(d0_ref[...] + d1_ref[...], 1.0)
    agg = (p0_ref[...] + p1_ref[...]) / deg + bc_ref[...]
    m = jax.nn.relu(agg)
    h = h_ref[...]
    gi = (
        jnp.dot(m, wi_ref[...], preferred_element_type=jnp.float32)
        + bi_ref[...]
    )
    gh = (
        jnp.dot(h, wh_ref[...], preferred_element_type=jnp.float32)
        + bh_ref[...]
    )
    r = jax.nn.sigmoid(gi[:, :DIM] + gh[:, :DIM])
    z = jax.nn.sigmoid(gi[:, DIM:2 * DIM] + gh[:, DIM:2 * DIM])
    n = jnp.tanh(gi[:, 2 * DIM:] + r * gh[:, 2 * DIM:])
    o_ref[...] = (1.0 - z) * n + z * h


def _gru(p0, p1, d0, d1, h, bcr, wit, bir, wht, bhr):
    return pl.pallas_call(
        _gru_body,
        grid=(N // TN_GRU,),
        in_specs=[
            pl.BlockSpec((TN_GRU, DIM), lambda i: (i, 0)),
            pl.BlockSpec((TN_GRU, DIM), lambda i: (i, 0)),
            pl.BlockSpec((TN_GRU, 1), lambda i: (i, 0)),
            pl.BlockSpec((TN_GRU, 1), lambda i: (i, 0)),
            pl.BlockSpec((TN_GRU, DIM), lambda i: (i, 0)),
            pl.BlockSpec((1, DIM), lambda i: (0, 0)),
            pl.BlockSpec((DIM, 3 * DIM), lambda i: (0, 0)),
            pl.BlockSpec((1, 3 * DIM), lambda i: (0, 0)),
            pl.BlockSpec((DIM, 3 * DIM), lambda i: (0, 0)),
            pl.BlockSpec((1, 3 * DIM), lambda i: (0, 0)),
        ],
        out_specs=pl.BlockSpec((TN_GRU, DIM), lambda i: (i, 0)),
        out_shape=jax.ShapeDtypeStruct((N, DIM), jnp.float32),
    )(p0, p1, d0, d1, h, bcr, wit, bir, wht, bhr)


def _s2s_body(h_ref, b_ref, wi_ref, wh_ref, bi_ref, bh_ref, o_ref):
    h = h_ref[...]
    seg = lax.broadcasted_iota(jnp.int32, (1, B), 1)
    onehot = b_ref[...] == seg
    ohf = onehot.astype(jnp.float32)
    neg_inf = jnp.float32(float("-inf"))

    q_star = jnp.zeros((B, 2 * DIM), jnp.float32)
    hl = jnp.zeros((B, DIM), jnp.float32)
    cl = jnp.zeros((B, DIM), jnp.float32)
    for _ in range(3):
        gates = (
            jnp.dot(q_star, wi_ref[...], preferred_element_type=jnp.float32)
            + bi_ref[...]
            + jnp.dot(hl, wh_ref[...], preferred_element_type=jnp.float32)
            + bh_ref[...]
        )
        ig = jax.nn.sigmoid(gates[:, :DIM])
        fg = jax.nn.sigmoid(gates[:, DIM:2 * DIM])
        gg = jnp.tanh(gates[:, 2 * DIM:3 * DIM])
        og = jax.nn.sigmoid(gates[:, 3 * DIM:])
        cl = fg * cl + ig * gg
        hl = og * jnp.tanh(cl)
        q = hl
        qb = jnp.dot(ohf, q, preferred_element_type=jnp.float32)
        e = jnp.sum(h * qb, axis=1, keepdims=True)
        em = jnp.max(jnp.where(onehot, e, neg_inf), axis=0, keepdims=True)
        em = jnp.where(em > neg_inf, em, 0.0)
        emb = jnp.sum(ohf * em, axis=1, keepdims=True)
        ex = jnp.exp(e - emb)
        es = jnp.sum(ohf * ex, axis=0, keepdims=True)
        esb = jnp.sum(ohf * es, axis=1, keepdims=True)
        a = ex / jnp.maximum(esb, 1e-16)
        rr = lax.dot_general(
            ohf, a * h, (((0,), (0,)), ((), ())),
            preferred_element_type=jnp.float32,
        )
        q_star = jnp.concatenate([q, rr], axis=1)
    o_ref[...] = q_star


def _s2s(h, batch2, wit, wht, bir, bhr):
    return pl.pallas_call(
        _s2s_body,
        grid=(1,),
        in_specs=[
            pl.BlockSpec((N, DIM), lambda i: (0, 0)),
            pl.BlockSpec((N, 1), lambda i: (0, 0)),
            pl.BlockSpec((2 * DIM, 4 * DIM), lambda i: (0, 0)),
            pl.BlockSpec((DIM, 4 * DIM), lambda i: (0, 0)),
            pl.BlockSpec((1, 4 * DIM), lambda i: (0, 0)),
            pl.BlockSpec((1, 4 * DIM), lambda i: (0, 0)),
        ],
        out_specs=pl.BlockSpec((B, 2 * DIM), lambda i: (0, 0)),
        out_shape=jax.ShapeDtypeStruct((B, 2 * DIM), jnp.float32),
    )(h, batch2, wit, wht, bir, bhr)


def kernel(x, edge_index, edge_attr, batch, W0, b0, We1, be1, We2, be2, bconv,
           Wih_g, Whh_g, bih_g, bhh_g, Wih_l, Whh_l, bih_l, bhh_l):
    src = edge_index[0]
    dst = edge_index[1]

    w0t = W0.T
    b0r = b0.reshape(1, DIM)
    w1t = We1.T
    b1r = be1.reshape(1, F_IN)
    w2t = We2.T
    bcr = bconv.reshape(1, DIM)
    wigt = Wih_g.T
    bigr = bih_g.reshape(1, 3 * DIM)
    whgt = Whh_g.T
    bhgr = bhh_g.reshape(1, 3 * DIM)
    wilt = Wih_l.T
    bilr = bih_l.reshape(1, 4 * DIM)
    whlt = Whh_l.T
    bhlr = bhh_l.reshape(1, 4 * DIM)
    rmat = (
        (jnp.arange(DIM * DIM) // DIM) == jnp.arange(DIM)[:, None]
    ).astype(jnp.bfloat16)
    w2tb = w2t.astype(jnp.bfloat16)
    b2b = be2.astype(jnp.bfloat16).reshape(1, DIM * DIM)

    zeros_nd = jnp.zeros((N, DIM), jnp.float32)
    zeros_n = jnp.zeros((N,), jnp.float32)
    ones_ch = jnp.ones((CH,), jnp.float32)

    h = _proj(x, w0t, b0r)

    d0 = d1 = None
    for it in range(3):
        xj = _gather_call(h, src)
        msg = _msg(edge_attr, xj, w1t, b1r, w2tb, b2b, rmat)
        if it == 0:
            part, degp = _scatter_deg_call(msg, dst, ones_ch, zeros_nd,
                                           zeros_n)
            d0 = degp[0].reshape(N, 1)
            d1 = degp[1].reshape(N, 1)
        else:
            part = _scatter_call(msg, dst, zeros_nd)
        h = _gru(part[0], part[1], d0, d1, h, bcr, wigt, bigr, whgt, bhgr)

    q_star = _s2s(h, batch.reshape(N, 1), wilt, whlt, bilr, bhlr)
    return q_star, h
